# Initial kernel scaffold; baseline (speedup 1.0000x reference)
import functools
import math

import jax
import jax.numpy as jnp
from jax.experimental import pallas as pl
from jax.experimental.pallas import tpu as pltpu

NUM_EMB = 1024
SPARSITY = 4
N_BINS = 16
COEF_MAX = 3.0
MU = 50.0
COMMIT = 0.25
EPS = 1e-10
LOG1P_MU = math.log1p(MU)


def _ldl_solve(gram, rhs):
    m = len(rhs)
    L = {}
    Dd = []
    for j in range(m):
        dj = gram[(j, j)] + 1e-8
        for p in range(j):
            dj = dj - L[(j, p)] * L[(j, p)] * Dd[p]
        Dd.append(dj)
        inv_dj = 1.0 / dj
        for i in range(j + 1, m):
            s = gram[(i, j)]
            for p in range(j):
                s = s - L[(i, p)] * L[(j, p)] * Dd[p]
            L[(i, j)] = s * inv_dj
    y = []
    for i in range(m):
        s = rhs[i]
        for j in range(i):
            s = s - L[(i, j)] * y[j]
        y.append(s)
    z = [y[i] / Dd[i] for i in range(m)]
    x = [None] * m
    for i in reversed(range(m)):
        s = z[i]
        for j in range(i + 1, m):
            s = s - L[(j, i)] * x[j]
        x[i] = s
    return x


def _omp_body(nb, denom, x_ref, dict_ref, zq_ref, tok_ref, loss_ref, dn_ref):
    pid = pl.program_id(0)
    f32 = jnp.float32
    dots = functools.partial(jax.lax.dot_general, preferred_element_type=f32)

    @pl.when(pid == 0)
    def _():
        d = dict_ref[...]
        n = jnp.sqrt(jnp.sum(d * d, axis=0, keepdims=True))
        dn_ref[...] = d / jnp.maximum(n, EPS)

    dn = dn_ref[...]
    x = x_ref[0]
    bt = x.shape[1]
    alpha0 = dots(dn, x, (((0,), (0,)), ((), ())))
    iota_n = jax.lax.broadcasted_iota(jnp.int32, (NUM_EMB, bt), 0)
    masked = jnp.zeros((NUM_EMB, bt), dtype=jnp.bool_)
    corr = alpha0
    atoms, rhs, sels = [], [], []
    gram = {}
    coeffs = None
    for k in range(SPARSITY):
        absc = jnp.where(masked, -1.0, jnp.abs(corr))
        mx = jnp.max(absc, axis=0, keepdims=True)
        sel = jnp.min(jnp.where(absc == mx, iota_n, NUM_EMB),
                      axis=0, keepdims=True)
        onehot = iota_n == sel
        masked = jnp.logical_or(masked, onehot)
        sels.append(sel)
        a_k = dots(dn, onehot.astype(f32), (((1,), (0,)), ((), ())))
        atoms.append(a_k)
        rhs.append(jnp.sum(jnp.where(onehot, alpha0, 0.0),
                           axis=0, keepdims=True))
        for j in range(k + 1):
            gram[(k, j)] = jnp.sum(a_k * atoms[j], axis=0, keepdims=True)
        coeffs = _ldl_solve(gram, rhs)
        if k < SPARSITY - 1:
            recon = functools.reduce(
                lambda a, b: a + b, [c * a for c, a in zip(coeffs, atoms)])
            corr = alpha0 - dots(dn, recon, (((0,), (0,)), ((), ())))

    toks = []
    recon_q = jnp.zeros_like(x)
    for k in range(SPARSITY):
        c = jnp.clip(coeffs[k], -COEF_MAX, COEF_MAX) / COEF_MAX
        enc = jnp.sign(c) * (jnp.log(1.0 + jnp.abs(c) * MU) / LOG1P_MU)
        scaled = (enc + 1.0) * ((N_BINS - 1) / 2.0)
        binf = jnp.clip(jnp.round(scaled), 0.0, float(N_BINS - 1))
        z = binf * (2.0 / (N_BINS - 1)) - 1.0
        cq = jnp.sign(z) * ((jnp.exp(jnp.abs(z) * LOG1P_MU) - 1.0) / MU) * COEF_MAX
        recon_q = recon_q + cq * atoms[k]
        toks.append(sels[k] * N_BINS + binf.astype(jnp.int32))

    zq_ref[0] = x + (recon_q - x)
    tok_ref[...] = jnp.concatenate(toks, axis=0)
    diff = recon_q - x
    sse = jnp.sum(diff * diff)
    prev = jnp.where(pid == 0, 0.0, loss_ref[0, 0])
    tot = prev + sse
    loss_ref[0, 0] = jnp.where(pid == nb - 1,
                               tot * ((1.0 + COMMIT) / denom), tot)


def kernel(z_e, dictionary):
    Bz, C, H, W = z_e.shape
    HW = H * W
    total = Bz * HW
    Bt = min(512, HW)
    pb = HW // Bt
    nb = total // Bt
    x3 = z_e.reshape(Bz, C, HW)
    body = functools.partial(_omp_body, nb, float(z_e.size))
    zq3, tok, loss = pl.pallas_call(
        body,
        grid=(nb,),
        in_specs=[
            pl.BlockSpec((1, C, Bt), lambda i: (i // pb, 0, i % pb)),
            pl.BlockSpec((C, NUM_EMB), lambda i: (0, 0)),
        ],
        out_specs=[
            pl.BlockSpec((1, C, Bt), lambda i: (i // pb, 0, i % pb)),
            pl.BlockSpec((SPARSITY, Bt), lambda i: (0, i)),
            pl.BlockSpec((1, 1), lambda i: (0, 0)),
        ],
        out_shape=[
            jax.ShapeDtypeStruct((Bz, C, HW), jnp.float32),
            jax.ShapeDtypeStruct((SPARSITY, total), jnp.int32),
            jax.ShapeDtypeStruct((1, 1), jnp.float32),
        ],
        scratch_shapes=[pltpu.VMEM((C, NUM_EMB), jnp.float32)],
    )(x3, dictionary)
    z_q_ste = zq3.reshape(Bz, C, H, W)
    tokens = tok.T.reshape(Bz, H, W, SPARSITY)
    return z_q_ste, loss[0, 0], tokens


# speedup vs baseline: 57.6047x; 57.6047x over previous
import functools
import math

import jax
import jax.numpy as jnp
from jax.experimental import pallas as pl
from jax.experimental.pallas import tpu as pltpu

NUM_EMB = 1024
SPARSITY = 4
N_BINS = 16
COEF_MAX = 3.0
MU = 50.0
COMMIT = 0.25
EPS = 1e-10
LOG1P_MU = math.log1p(MU)
BF16 = jnp.bfloat16
F32 = jnp.float32


def _mm(a, b, dims):
    return jax.lax.dot_general(a, b, (dims, ((), ())),
                               preferred_element_type=F32)


def _ldl_solve(gram, rhs):
    m = len(rhs)
    L = {}
    Dd = []
    for j in range(m):
        dj = gram[(j, j)] + 1e-8
        for p in range(j):
            dj = dj - L[(j, p)] * L[(j, p)] * Dd[p]
        Dd.append(dj)
        inv_dj = 1.0 / dj
        for i in range(j + 1, m):
            s = gram[(i, j)]
            for p in range(j):
                s = s - L[(i, p)] * L[(j, p)] * Dd[p]
            L[(i, j)] = s * inv_dj
    y = []
    for i in range(m):
        s = rhs[i]
        for j in range(i):
            s = s - L[(i, j)] * y[j]
        y.append(s)
    z = [y[i] / Dd[i] for i in range(m)]
    x = [None] * m
    for i in reversed(range(m)):
        s = z[i]
        for j in range(i + 1, m):
            s = s - L[(j, i)] * x[j]
        x[i] = s
    return x


def _omp_body(nb, denom,
              x_ref, dict_ref,
              zq_ref, tok_ref, loss_ref,
              dnh_ref, dnl_ref, gh_ref, gm_ref, gl_ref, diag_ref):
    pid = pl.program_id(0)

    @pl.when(pid == 0)
    def _():
        d = dict_ref[...]
        n = jnp.sqrt(jnp.sum(d * d, axis=0, keepdims=True))
        dn = d / jnp.maximum(n, EPS)
        dn_h = dn.astype(BF16)
        dnh_ref[...] = dn_h
        dnl_ref[...] = (dn - dn_h.astype(F32)).astype(BF16)
        g = _mm(dn_h, dn_h, (((0,), (0,))))
        g_h = g.astype(BF16)
        r1 = g - g_h.astype(F32)
        g_m = r1.astype(BF16)
        gh_ref[...] = g_h
        gm_ref[...] = g_m
        gl_ref[...] = (r1 - g_m.astype(F32)).astype(BF16)
        ii = jax.lax.broadcasted_iota(jnp.int32, (NUM_EMB, NUM_EMB), 0)
        jj = jax.lax.broadcasted_iota(jnp.int32, (NUM_EMB, NUM_EMB), 1)
        diag_ref[...] = jnp.sum(jnp.where(ii == jj, g, 0.0),
                                axis=1, keepdims=True)

    x = x_ref[0]
    bt = x.shape[1]
    dn_h = dnh_ref[...]
    alpha0 = _mm(dn_h, x.astype(BF16), (((0,), (0,))))
    iota_n = jax.lax.broadcasted_iota(jnp.int32, (NUM_EMB, bt), 0)
    masked = jnp.zeros((NUM_EMB, bt), dtype=jnp.bool_)
    corr = alpha0
    onehots, gcols, rhs, sels = [], [], [], []
    gram = {}
    coeffs = None
    for k in range(SPARSITY):
        absc = jnp.where(masked, -1.0, jnp.abs(corr))
        mx = jnp.max(absc, axis=0, keepdims=True)
        sel = jnp.min(jnp.where(absc == mx, iota_n, NUM_EMB),
                      axis=0, keepdims=True)
        onehot = iota_n == sel
        masked = jnp.logical_or(masked, onehot)
        sels.append(sel)
        onehots.append(onehot)
        rhs.append(jnp.sum(jnp.where(onehot, alpha0, 0.0),
                           axis=0, keepdims=True))
        if k < SPARSITY - 1:
            oh = onehot.astype(BF16)
            gcol = (_mm(gh_ref[...], oh, (((1,), (0,))))
                    + _mm(gm_ref[...], oh, (((1,), (0,))))
                    + _mm(gl_ref[...], oh, (((1,), (0,)))))
            gcols.append(gcol)
            for j in range(k + 1):
                gram[(k, j)] = jnp.sum(jnp.where(onehot, gcols[j], 0.0),
                                       axis=0, keepdims=True)
        else:
            for j in range(k):
                gram[(k, j)] = jnp.sum(jnp.where(onehot, gcols[j], 0.0),
                                       axis=0, keepdims=True)
            gram[(k, k)] = jnp.sum(
                jnp.where(onehot, diag_ref[...], 0.0), axis=0, keepdims=True)
        coeffs = _ldl_solve(gram, rhs)
        if k < SPARSITY - 1:
            delta = coeffs[0] * gcols[0]
            for j in range(1, k + 1):
                delta = delta + coeffs[j] * gcols[j]
            corr = alpha0 - delta
    dn_l = dnl_ref[...]
    toks = []
    recon_q = jnp.zeros_like(x)
    for k in range(SPARSITY):
        c = jnp.clip(coeffs[k], -COEF_MAX, COEF_MAX) / COEF_MAX
        enc = jnp.sign(c) * (jnp.log1p(jnp.abs(c) * MU) / LOG1P_MU)
        scaled = (enc + 1.0) * ((N_BINS - 1) / 2.0)
        binf = jnp.clip(jnp.round(scaled), 0.0, float(N_BINS - 1))
        z = binf * (2.0 / (N_BINS - 1)) - 1.0
        cq = jnp.sign(z) * ((jnp.exp(jnp.abs(z) * LOG1P_MU) - 1.0) / MU) * COEF_MAX
        oh = onehots[k].astype(BF16)
        a_k = (_mm(dn_h, oh, (((1,), (0,))))
               + _mm(dn_l, oh, (((1,), (0,)))))
        recon_q = recon_q + cq * a_k
        toks.append(sels[k] * N_BINS + binf.astype(jnp.int32))

    zq_ref[0] = x + (recon_q - x)
    tok_ref[...] = jnp.concatenate(toks, axis=0)
    diff = recon_q - x
    sse = jnp.sum(diff * diff)
    prev = jnp.where(pid == 0, jnp.zeros((1, 1), F32), loss_ref[...])
    tot = prev + sse
    loss_ref[...] = jnp.where(pid == nb - 1,
                              tot * ((1.0 + COMMIT) / denom), tot)


def kernel(z_e, dictionary):
    Bz, C, H, W = z_e.shape
    HW = H * W
    total = Bz * HW
    Bt = min(512, HW)
    pb = HW // Bt
    nb = total // Bt
    x3 = z_e.reshape(Bz, C, HW)
    body = functools.partial(_omp_body, nb, float(z_e.size))
    zq3, tok, loss = pl.pallas_call(
        body,
        grid=(nb,),
        in_specs=[
            pl.BlockSpec((1, C, Bt), lambda i: (i // pb, 0, i % pb)),
            pl.BlockSpec((C, NUM_EMB), lambda i: (0, 0)),
        ],
        out_specs=[
            pl.BlockSpec((1, C, Bt), lambda i: (i // pb, 0, i % pb)),
            pl.BlockSpec((SPARSITY, Bt), lambda i: (0, i)),
            pl.BlockSpec((1, 1), lambda i: (0, 0)),
        ],
        out_shape=[
            jax.ShapeDtypeStruct((Bz, C, HW), jnp.float32),
            jax.ShapeDtypeStruct((SPARSITY, total), jnp.int32),
            jax.ShapeDtypeStruct((1, 1), jnp.float32),
        ],
        scratch_shapes=[
            pltpu.VMEM((C, NUM_EMB), BF16),
            pltpu.VMEM((C, NUM_EMB), BF16),
            pltpu.VMEM((NUM_EMB, NUM_EMB), BF16),
            pltpu.VMEM((NUM_EMB, NUM_EMB), BF16),
            pltpu.VMEM((NUM_EMB, NUM_EMB), BF16),
            pltpu.VMEM((NUM_EMB, 1), F32),
        ],
    )(x3, dictionary)
    z_q_ste = zq3.reshape(Bz, C, H, W)
    tokens = tok.T.reshape(Bz, H, W, SPARSITY)
    return z_q_ste, loss[0, 0], tokens


# speedup vs baseline: 80.6332x; 1.3998x over previous
import functools
import math

import jax
import jax.numpy as jnp
from jax.experimental import pallas as pl
from jax.experimental.pallas import tpu as pltpu

NUM_EMB = 1024
SPARSITY = 4
N_BINS = 16
COEF_MAX = 3.0
MU = 50.0
COMMIT = 0.25
EPS = 1e-10
LOG1P_MU = math.log1p(MU)
BF16 = jnp.bfloat16
F32 = jnp.float32


def _mm(a, b, dims):
    return jax.lax.dot_general(a, b, (dims, ((), ())),
                               preferred_element_type=F32)


def _ldl_solve(gram, rhs):
    m = len(rhs)
    L = {}
    Dd = []
    for j in range(m):
        dj = gram[(j, j)] + 1e-8
        for p in range(j):
            dj = dj - L[(j, p)] * L[(j, p)] * Dd[p]
        Dd.append(dj)
        inv_dj = 1.0 / dj
        for i in range(j + 1, m):
            s = gram[(i, j)]
            for p in range(j):
                s = s - L[(i, p)] * L[(j, p)] * Dd[p]
            L[(i, j)] = s * inv_dj
    y = []
    for i in range(m):
        s = rhs[i]
        for j in range(i):
            s = s - L[(i, j)] * y[j]
        y.append(s)
    z = [y[i] / Dd[i] for i in range(m)]
    x = [None] * m
    for i in reversed(range(m)):
        s = z[i]
        for j in range(i + 1, m):
            s = s - L[(j, i)] * x[j]
        x[i] = s
    return x


def _omp_body(nb, denom,
              x_ref, dict_ref,
              zq_ref, tok_ref, loss_ref,
              dnh_ref, dnl_ref, gh_ref, gm_ref):
    pid = pl.program_id(0)

    @pl.when(pid == 0)
    def _():
        d = dict_ref[...]
        n = jnp.sqrt(jnp.sum(d * d, axis=0, keepdims=True))
        dn = d / jnp.maximum(n, EPS)
        dn_h = dn.astype(BF16)
        dnh_ref[...] = dn_h
        dnl_ref[...] = (dn - dn_h.astype(F32)).astype(BF16)
        g = _mm(dn_h, dn_h, (((0,), (0,))))
        g_h = g.astype(BF16)
        gh_ref[...] = g_h
        gm_ref[...] = (g - g_h.astype(F32)).astype(BF16)

    x = x_ref[0]
    bt = x.shape[1]
    dn_h = dnh_ref[...]
    x_bf = x.astype(BF16)
    alpha0 = _mm(dn_h, x_bf, (((0,), (0,))))
    x_bf32 = x_bf.astype(F32)
    iota_n = jax.lax.broadcasted_iota(jnp.int32, (NUM_EMB, bt), 0)
    masked = jnp.zeros((NUM_EMB, bt), dtype=jnp.bool_)
    corr = alpha0
    onehots, gcols, atoms_h, rhs, sels = [], [], [], [], []
    gram = {}
    coeffs = None
    for k in range(SPARSITY):
        absc = jnp.where(masked, -1.0, jnp.abs(corr))
        mx = jnp.max(absc, axis=0, keepdims=True)
        sel = jnp.min(jnp.where(absc == mx, iota_n, NUM_EMB),
                      axis=0, keepdims=True)
        onehot = iota_n == sel
        masked = jnp.logical_or(masked, onehot)
        sels.append(sel)
        onehots.append(onehot)
        oh = onehot.astype(BF16)
        a_h = _mm(dn_h, oh, (((1,), (0,))))
        atoms_h.append(a_h)
        rhs.append(jnp.sum(a_h * x_bf32, axis=0, keepdims=True))
        for j in range(k + 1):
            gram[(k, j)] = jnp.sum(a_h * atoms_h[j], axis=0, keepdims=True)
        coeffs = _ldl_solve(gram, rhs)
        if k < SPARSITY - 1:
            gcol = (_mm(gh_ref[...], oh, (((1,), (0,))))
                    + _mm(gm_ref[...], oh, (((1,), (0,)))))
            gcols.append(gcol)
            delta = coeffs[0] * gcols[0]
            for j in range(1, k + 1):
                delta = delta + coeffs[j] * gcols[j]
            corr = alpha0 - delta
    dn_l = dnl_ref[...]
    toks = []
    recon_q = jnp.zeros_like(x)
    for k in range(SPARSITY):
        c = jnp.clip(coeffs[k], -COEF_MAX, COEF_MAX) / COEF_MAX
        enc = jnp.sign(c) * (jnp.log1p(jnp.abs(c) * MU) / LOG1P_MU)
        scaled = (enc + 1.0) * ((N_BINS - 1) / 2.0)
        binf = jnp.clip(jnp.round(scaled), 0.0, float(N_BINS - 1))
        z = binf * (2.0 / (N_BINS - 1)) - 1.0
        cq = jnp.sign(z) * ((jnp.exp(jnp.abs(z) * LOG1P_MU) - 1.0) / MU) * COEF_MAX
        a_k = atoms_h[k] + _mm(dn_l, onehots[k].astype(BF16), (((1,), (0,))))
        recon_q = recon_q + cq * a_k
        toks.append(sels[k] * N_BINS + binf.astype(jnp.int32))

    zq_ref[0] = x + (recon_q - x)
    tok_ref[...] = jnp.concatenate(toks, axis=0)
    diff = recon_q - x
    sse = jnp.sum(diff * diff)
    prev = jnp.where(pid == 0, jnp.zeros((1, 1), F32), loss_ref[...])
    tot = prev + sse
    loss_ref[...] = jnp.where(pid == nb - 1,
                              tot * ((1.0 + COMMIT) / denom), tot)


def kernel(z_e, dictionary):
    Bz, C, H, W = z_e.shape
    HW = H * W
    total = Bz * HW
    Bt = min(512, HW)
    pb = HW // Bt
    nb = total // Bt
    x3 = z_e.reshape(Bz, C, HW)
    body = functools.partial(_omp_body, nb, float(z_e.size))
    zq3, tok, loss = pl.pallas_call(
        body,
        grid=(nb,),
        in_specs=[
            pl.BlockSpec((1, C, Bt), lambda i: (i // pb, 0, i % pb)),
            pl.BlockSpec((C, NUM_EMB), lambda i: (0, 0)),
        ],
        out_specs=[
            pl.BlockSpec((1, C, Bt), lambda i: (i // pb, 0, i % pb)),
            pl.BlockSpec((SPARSITY, Bt), lambda i: (0, i)),
            pl.BlockSpec((1, 1), lambda i: (0, 0)),
        ],
        out_shape=[
            jax.ShapeDtypeStruct((Bz, C, HW), jnp.float32),
            jax.ShapeDtypeStruct((SPARSITY, total), jnp.int32),
            jax.ShapeDtypeStruct((1, 1), jnp.float32),
        ],
        scratch_shapes=[
            pltpu.VMEM((C, NUM_EMB), BF16),
            pltpu.VMEM((C, NUM_EMB), BF16),
            pltpu.VMEM((NUM_EMB, NUM_EMB), BF16),
            pltpu.VMEM((NUM_EMB, NUM_EMB), BF16),
        ],
    )(x3, dictionary)
    z_q_ste = zq3.reshape(Bz, C, H, W)
    tokens = tok.T.reshape(Bz, H, W, SPARSITY)
    return z_q_ste, loss[0, 0], tokens


# speedup vs baseline: 82.2385x; 1.0199x over previous
import functools
import math

import jax
import jax.numpy as jnp
from jax.experimental import pallas as pl
from jax.experimental.pallas import tpu as pltpu

NUM_EMB = 1024
SPARSITY = 4
N_BINS = 16
COEF_MAX = 3.0
MU = 50.0
COMMIT = 0.25
EPS = 1e-10
LOG1P_MU = math.log1p(MU)
BF16 = jnp.bfloat16
F32 = jnp.float32


def _mm(a, b, dims):
    return jax.lax.dot_general(a, b, (dims, ((), ())),
                               preferred_element_type=F32)


def _ldl_solve(gram, rhs):
    m = len(rhs)
    L = {}
    Dd = []
    for j in range(m):
        dj = gram[(j, j)] + 1e-8
        for p in range(j):
            dj = dj - L[(j, p)] * L[(j, p)] * Dd[p]
        Dd.append(dj)
        inv_dj = 1.0 / dj
        for i in range(j + 1, m):
            s = gram[(i, j)]
            for p in range(j):
                s = s - L[(i, p)] * L[(j, p)] * Dd[p]
            L[(i, j)] = s * inv_dj
    y = []
    for i in range(m):
        s = rhs[i]
        for j in range(i):
            s = s - L[(i, j)] * y[j]
        y.append(s)
    z = [y[i] / Dd[i] for i in range(m)]
    x = [None] * m
    for i in reversed(range(m)):
        s = z[i]
        for j in range(i + 1, m):
            s = s - L[(j, i)] * x[j]
        x[i] = s
    return x


def _omp_half(dn_h, dn_l, gh, gm, x):
    bt = x.shape[1]
    x_bf = x.astype(BF16)
    alpha0 = _mm(dn_h, x_bf, (((0,), (0,))))
    x_bf32 = x_bf.astype(F32)
    iota_n = jax.lax.broadcasted_iota(jnp.int32, (NUM_EMB, bt), 0)
    masked = jnp.zeros((NUM_EMB, bt), dtype=jnp.bool_)
    corr = alpha0
    onehots, gcols, atoms_h, rhs, sels = [], [], [], [], []
    gram = {}
    coeffs = None
    for k in range(SPARSITY):
        absc = jnp.where(masked, -1.0, jnp.abs(corr))
        mx = jnp.max(absc, axis=0, keepdims=True)
        sel = jnp.min(jnp.where(absc == mx, iota_n, NUM_EMB),
                      axis=0, keepdims=True)
        onehot = iota_n == sel
        masked = jnp.logical_or(masked, onehot)
        sels.append(sel)
        onehots.append(onehot)
        oh = onehot.astype(BF16)
        a_h = _mm(dn_h, oh, (((1,), (0,))))
        atoms_h.append(a_h)
        rhs.append(jnp.sum(a_h * x_bf32, axis=0, keepdims=True))
        for j in range(k + 1):
            gram[(k, j)] = jnp.sum(a_h * atoms_h[j], axis=0, keepdims=True)
        coeffs = _ldl_solve(gram, rhs)
        if k < SPARSITY - 1:
            gcol = (_mm(gh, oh, (((1,), (0,))))
                    + _mm(gm, oh, (((1,), (0,)))))
            gcols.append(gcol)
            delta = coeffs[0] * gcols[0]
            for j in range(1, k + 1):
                delta = delta + coeffs[j] * gcols[j]
            corr = alpha0 - delta
    toks = []
    recon_q = jnp.zeros_like(x)
    for k in range(SPARSITY):
        c = jnp.clip(coeffs[k], -COEF_MAX, COEF_MAX) / COEF_MAX
        enc = jnp.sign(c) * (jnp.log1p(jnp.abs(c) * MU) / LOG1P_MU)
        scaled = (enc + 1.0) * ((N_BINS - 1) / 2.0)
        binf = jnp.clip(jnp.round(scaled), 0.0, float(N_BINS - 1))
        z = binf * (2.0 / (N_BINS - 1)) - 1.0
        cq = jnp.sign(z) * ((jnp.exp(jnp.abs(z) * LOG1P_MU) - 1.0) / MU) * COEF_MAX
        a_k = atoms_h[k] + _mm(dn_l, onehots[k].astype(BF16), (((1,), (0,))))
        recon_q = recon_q + cq * a_k
        toks.append(sels[k] * N_BINS + binf.astype(jnp.int32))
    diff = recon_q - x
    sse = jnp.sum(diff * diff)
    return x + (recon_q - x), jnp.concatenate(toks, axis=0), sse


def _omp_body(nb, denom, nh,
              x_ref, dict_ref,
              zq_ref, tok_ref, loss_ref,
              dnh_ref, dnl_ref, gh_ref, gm_ref):
    pid = pl.program_id(0)

    @pl.when(pid == 0)
    def _():
        d = dict_ref[...]
        n = jnp.sqrt(jnp.sum(d * d, axis=0, keepdims=True))
        dn = d / jnp.maximum(n, EPS)
        dn_h = dn.astype(BF16)
        dnh_ref[...] = dn_h
        dnl_ref[...] = (dn - dn_h.astype(F32)).astype(BF16)
        g = _mm(dn_h, dn_h, (((0,), (0,))))
        g_h = g.astype(BF16)
        gh_ref[...] = g_h
        gm_ref[...] = (g - g_h.astype(F32)).astype(BF16)

    x = x_ref[0]
    bt = x.shape[1]
    hw = bt // nh
    dn_h = dnh_ref[...]
    dn_l = dnl_ref[...]
    gh = gh_ref[...]
    gm = gm_ref[...]
    sse = None
    for h in range(nh):
        zq_h, tok_h, sse_h = _omp_half(dn_h, dn_l, gh, gm,
                                       x[:, h * hw:(h + 1) * hw])
        zq_ref[0, :, h * hw:(h + 1) * hw] = zq_h
        tok_ref[:, h * hw:(h + 1) * hw] = tok_h
        sse = sse_h if sse is None else sse + sse_h
    prev = jnp.where(pid == 0, jnp.zeros((1, 1), F32), loss_ref[...])
    tot = prev + sse
    loss_ref[...] = jnp.where(pid == nb - 1,
                              tot * ((1.0 + COMMIT) / denom), tot)


def kernel(z_e, dictionary):
    Bz, C, H, W = z_e.shape
    HW = H * W
    total = Bz * HW
    Bt = min(1024, HW)
    nh = 2
    pb = HW // Bt
    nb = total // Bt
    x3 = z_e.reshape(Bz, C, HW)
    body = functools.partial(_omp_body, nb, float(z_e.size), nh)
    zq3, tok, loss = pl.pallas_call(
        body,
        grid=(nb,),
        in_specs=[
            pl.BlockSpec((1, C, Bt), lambda i: (i // pb, 0, i % pb)),
            pl.BlockSpec((C, NUM_EMB), lambda i: (0, 0)),
        ],
        out_specs=[
            pl.BlockSpec((1, C, Bt), lambda i: (i // pb, 0, i % pb)),
            pl.BlockSpec((SPARSITY, Bt), lambda i: (0, i)),
            pl.BlockSpec((1, 1), lambda i: (0, 0)),
        ],
        out_shape=[
            jax.ShapeDtypeStruct((Bz, C, HW), jnp.float32),
            jax.ShapeDtypeStruct((SPARSITY, total), jnp.int32),
            jax.ShapeDtypeStruct((1, 1), jnp.float32),
        ],
        scratch_shapes=[
            pltpu.VMEM((C, NUM_EMB), BF16),
            pltpu.VMEM((C, NUM_EMB), BF16),
            pltpu.VMEM((NUM_EMB, NUM_EMB), BF16),
            pltpu.VMEM((NUM_EMB, NUM_EMB), BF16),
        ],
    )(x3, dictionary)
    z_q_ste = zq3.reshape(Bz, C, H, W)
    tokens = tok.T.reshape(Bz, H, W, SPARSITY)
    return z_q_ste, loss[0, 0], tokens


# speedup vs baseline: 86.2800x; 1.0491x over previous
import functools
import math

import jax
import jax.numpy as jnp
from jax.experimental import pallas as pl
from jax.experimental.pallas import tpu as pltpu

NUM_EMB = 1024
SPARSITY = 4
N_BINS = 16
COEF_MAX = 3.0
MU = 50.0
COMMIT = 0.25
EPS = 1e-10
LOG1P_MU = math.log1p(MU)
BF16 = jnp.bfloat16
F32 = jnp.float32


def _mm(a, b, dims):
    return jax.lax.dot_general(a, b, (dims, ((), ())),
                               preferred_element_type=F32)


def _ldl_solve(gram, rhs):
    m = len(rhs)
    L = {}
    Dd = []
    for j in range(m):
        dj = gram[(j, j)] + 1e-8
        for p in range(j):
            dj = dj - L[(j, p)] * L[(j, p)] * Dd[p]
        Dd.append(dj)
        inv_dj = 1.0 / dj
        for i in range(j + 1, m):
            s = gram[(i, j)]
            for p in range(j):
                s = s - L[(i, p)] * L[(j, p)] * Dd[p]
            L[(i, j)] = s * inv_dj
    y = []
    for i in range(m):
        s = rhs[i]
        for j in range(i):
            s = s - L[(i, j)] * y[j]
        y.append(s)
    z = [y[i] / Dd[i] for i in range(m)]
    x = [None] * m
    for i in reversed(range(m)):
        s = z[i]
        for j in range(i + 1, m):
            s = s - L[(j, i)] * x[j]
        x[i] = s
    return x


def _omp_slabs(dn_h, dn_l, gh, gm, xs):
    ns = len(xs)
    S = range(ns)
    bt = xs[0].shape[1]
    x_bf = [x.astype(BF16) for x in xs]
    alpha0 = [_mm(dn_h, xb, (((0,), (0,)))) for xb in x_bf]
    x_bf32 = [xb.astype(F32) for xb in x_bf]
    iota_n = jax.lax.broadcasted_iota(jnp.int32, (NUM_EMB, bt), 0)
    masked = [jnp.zeros((NUM_EMB, bt), dtype=jnp.bool_) for _ in S]
    corr = list(alpha0)
    onehots = [[] for _ in S]
    gcols = [[] for _ in S]
    atoms_h = [[] for _ in S]
    rhs = [[] for _ in S]
    sels = [[] for _ in S]
    gram = [{} for _ in S]
    coeffs = [None] * ns
    for k in range(SPARSITY):
        absc = [jnp.where(masked[s], -1.0, jnp.abs(corr[s])) for s in S]
        mx = [jnp.max(absc[s], axis=0, keepdims=True) for s in S]
        sel = [jnp.min(jnp.where(absc[s] == mx[s], iota_n, NUM_EMB),
                       axis=0, keepdims=True) for s in S]
        onehot = [iota_n == sel[s] for s in S]
        oh = [onehot[s].astype(BF16) for s in S]
        a_h = [_mm(dn_h, oh[s], (((1,), (0,)))) for s in S]
        for s in S:
            masked[s] = jnp.logical_or(masked[s], onehot[s])
            sels[s].append(sel[s])
            onehots[s].append(onehot[s])
            atoms_h[s].append(a_h[s])
            rhs[s].append(jnp.sum(a_h[s] * x_bf32[s], axis=0, keepdims=True))
            for j in range(k + 1):
                gram[s][(k, j)] = jnp.sum(a_h[s] * atoms_h[s][j],
                                          axis=0, keepdims=True)
        coeffs = [_ldl_solve(gram[s], rhs[s]) for s in S]
        if k < SPARSITY - 1:
            gcol_h = [_mm(gh, oh[s], (((1,), (0,)))) for s in S]
            gcol_m = [_mm(gm, oh[s], (((1,), (0,)))) for s in S]
            for s in S:
                gcols[s].append(gcol_h[s] + gcol_m[s])
                delta = coeffs[s][0] * gcols[s][0]
                for j in range(1, k + 1):
                    delta = delta + coeffs[s][j] * gcols[s][j]
                corr[s] = alpha0[s] - delta
    toks = [[] for _ in S]
    recon_q = [jnp.zeros_like(xs[s]) for s in S]
    for k in range(SPARSITY):
        a_l = [_mm(dn_l, onehots[s][k].astype(BF16), (((1,), (0,))))
               for s in S]
        for s in S:
            c = jnp.clip(coeffs[s][k], -COEF_MAX, COEF_MAX) / COEF_MAX
            enc = jnp.sign(c) * (jnp.log1p(jnp.abs(c) * MU) / LOG1P_MU)
            scaled = (enc + 1.0) * ((N_BINS - 1) / 2.0)
            binf = jnp.clip(jnp.round(scaled), 0.0, float(N_BINS - 1))
            z = binf * (2.0 / (N_BINS - 1)) - 1.0
            cq = (jnp.sign(z) * ((jnp.exp(jnp.abs(z) * LOG1P_MU) - 1.0) / MU)
                  * COEF_MAX)
            recon_q[s] = recon_q[s] + cq * (atoms_h[s][k] + a_l[s])
            toks[s].append(sels[s][k] * N_BINS + binf.astype(jnp.int32))
    sse = None
    zqs, tokcat = [], []
    for s in S:
        d = recon_q[s] - xs[s]
        sse_s = jnp.sum(d * d)
        sse = sse_s if sse is None else sse + sse_s
        zqs.append(xs[s] + (recon_q[s] - xs[s]))
        tokcat.append(jnp.concatenate(toks[s], axis=0))
    return zqs, tokcat, sse


def _omp_body(nb, denom, nh,
              x_ref, dict_ref,
              zq_ref, tok_ref, loss_ref,
              dnh_ref, dnl_ref, gh_ref, gm_ref):
    pid = pl.program_id(0)

    @pl.when(pid == 0)
    def _():
        d = dict_ref[...]
        n = jnp.sqrt(jnp.sum(d * d, axis=0, keepdims=True))
        dn = d / jnp.maximum(n, EPS)
        dn_h = dn.astype(BF16)
        dnh_ref[...] = dn_h
        dnl_ref[...] = (dn - dn_h.astype(F32)).astype(BF16)
        g = _mm(dn_h, dn_h, (((0,), (0,))))
        g_h = g.astype(BF16)
        gh_ref[...] = g_h
        gm_ref[...] = (g - g_h.astype(F32)).astype(BF16)

    x = x_ref[0]
    bt = x.shape[1]
    hw = bt // nh
    dn_h = dnh_ref[...]
    dn_l = dnl_ref[...]
    gh = gh_ref[...]
    gm = gm_ref[...]
    xs = [x[:, h * hw:(h + 1) * hw] for h in range(nh)]
    zqs, tokcat, sse = _omp_slabs(dn_h, dn_l, gh, gm, xs)
    for h in range(nh):
        zq_ref[0, :, h * hw:(h + 1) * hw] = zqs[h]
        tok_ref[:, h * hw:(h + 1) * hw] = tokcat[h]
    prev = jnp.where(pid == 0, jnp.zeros((1, 1), F32), loss_ref[...])
    tot = prev + sse
    loss_ref[...] = jnp.where(pid == nb - 1,
                              tot * ((1.0 + COMMIT) / denom), tot)


def kernel(z_e, dictionary):
    Bz, C, H, W = z_e.shape
    HW = H * W
    total = Bz * HW
    Bt = min(1024, HW)
    nh = 2
    pb = HW // Bt
    nb = total // Bt
    x3 = z_e.reshape(Bz, C, HW)
    body = functools.partial(_omp_body, nb, float(z_e.size), nh)
    zq3, tok, loss = pl.pallas_call(
        body,
        grid=(nb,),
        in_specs=[
            pl.BlockSpec((1, C, Bt), lambda i: (i // pb, 0, i % pb)),
            pl.BlockSpec((C, NUM_EMB), lambda i: (0, 0)),
        ],
        out_specs=[
            pl.BlockSpec((1, C, Bt), lambda i: (i // pb, 0, i % pb)),
            pl.BlockSpec((SPARSITY, Bt), lambda i: (0, i)),
            pl.BlockSpec((1, 1), lambda i: (0, 0)),
        ],
        out_shape=[
            jax.ShapeDtypeStruct((Bz, C, HW), jnp.float32),
            jax.ShapeDtypeStruct((SPARSITY, total), jnp.int32),
            jax.ShapeDtypeStruct((1, 1), jnp.float32),
        ],
        scratch_shapes=[
            pltpu.VMEM((C, NUM_EMB), BF16),
            pltpu.VMEM((C, NUM_EMB), BF16),
            pltpu.VMEM((NUM_EMB, NUM_EMB), BF16),
            pltpu.VMEM((NUM_EMB, NUM_EMB), BF16),
        ],
    )(x3, dictionary)
    z_q_ste = zq3.reshape(Bz, C, H, W)
    tokens = tok.T.reshape(Bz, H, W, SPARSITY)
    return z_q_ste, loss[0, 0], tokens


# speedup vs baseline: 86.9846x; 1.0082x over previous
import functools
import math

import jax
import jax.numpy as jnp
from jax.experimental import pallas as pl
from jax.experimental.pallas import tpu as pltpu

NUM_EMB = 1024
SPARSITY = 4
N_BINS = 16
COEF_MAX = 3.0
MU = 50.0
COMMIT = 0.25
EPS = 1e-10
LOG1P_MU = math.log1p(MU)
BF16 = jnp.bfloat16
F32 = jnp.float32


def _mm(a, b, dims):
    return jax.lax.dot_general(a, b, (dims, ((), ())),
                               preferred_element_type=F32)


def _ldl_solve(gram, rhs):
    m = len(rhs)
    L = {}
    Dd = []
    for j in range(m):
        dj = gram[(j, j)] + 1e-8
        for p in range(j):
            dj = dj - L[(j, p)] * L[(j, p)] * Dd[p]
        Dd.append(dj)
        inv_dj = 1.0 / dj
        for i in range(j + 1, m):
            s = gram[(i, j)]
            for p in range(j):
                s = s - L[(i, p)] * L[(j, p)] * Dd[p]
            L[(i, j)] = s * inv_dj
    y = []
    for i in range(m):
        s = rhs[i]
        for j in range(i):
            s = s - L[(i, j)] * y[j]
        y.append(s)
    z = [y[i] / Dd[i] for i in range(m)]
    x = [None] * m
    for i in reversed(range(m)):
        s = z[i]
        for j in range(i + 1, m):
            s = s - L[(j, i)] * x[j]
        x[i] = s
    return x


def _omp_slabs(dn_h, dn_l, gh, gm, xs):
    ns = len(xs)
    S = range(ns)
    bt = xs[0].shape[1]
    x_bf = [x.astype(BF16) for x in xs]
    alpha0 = [_mm(dn_h, xb, (((0,), (0,)))) for xb in x_bf]
    x_bf32 = [xb.astype(F32) for xb in x_bf]
    iota_n = jax.lax.broadcasted_iota(jnp.int32, (NUM_EMB, bt), 0)
    masked = [jnp.zeros((NUM_EMB, bt), dtype=jnp.bool_) for _ in S]
    corr = list(alpha0)
    onehots = [[] for _ in S]
    gcols = [[] for _ in S]
    atoms_h = [[] for _ in S]
    rhs = [[] for _ in S]
    sels = [[] for _ in S]
    gram = [{} for _ in S]
    coeffs = [None] * ns
    for k in range(SPARSITY):
        absc = [jnp.where(masked[s], -1.0, jnp.abs(corr[s])) for s in S]
        mx = [jnp.max(absc[s], axis=0, keepdims=True) for s in S]
        sel = [jnp.min(jnp.where(absc[s] == mx[s], iota_n, NUM_EMB),
                       axis=0, keepdims=True) for s in S]
        onehot = [iota_n == sel[s] for s in S]
        oh = [onehot[s].astype(BF16) for s in S]
        a_h = [_mm(dn_h, oh[s], (((1,), (0,)))) for s in S]
        for s in S:
            masked[s] = jnp.logical_or(masked[s], onehot[s])
            sels[s].append(sel[s])
            onehots[s].append(onehot[s])
            atoms_h[s].append(a_h[s])
            rhs[s].append(jnp.sum(a_h[s] * x_bf32[s], axis=0, keepdims=True))
            for j in range(k + 1):
                gram[s][(k, j)] = jnp.sum(a_h[s] * atoms_h[s][j],
                                          axis=0, keepdims=True)
        coeffs = [_ldl_solve(gram[s], rhs[s]) for s in S]
        if k < SPARSITY - 1:
            gcol_h = [_mm(gh, oh[s], (((1,), (0,)))) for s in S]
            gcol_m = [_mm(gm, oh[s], (((1,), (0,)))) for s in S]
            for s in S:
                gcols[s].append(gcol_h[s] + gcol_m[s])
                delta = coeffs[s][0] * gcols[s][0]
                for j in range(1, k + 1):
                    delta = delta + coeffs[s][j] * gcols[s][j]
                corr[s] = alpha0[s] - delta
    toks = [[] for _ in S]
    recon_q = [jnp.zeros_like(xs[s]) for s in S]
    for k in range(SPARSITY):
        a_l = [_mm(dn_l, onehots[s][k].astype(BF16), (((1,), (0,))))
               for s in S]
        for s in S:
            c = jnp.clip(coeffs[s][k], -COEF_MAX, COEF_MAX) / COEF_MAX
            enc = jnp.sign(c) * (jnp.log1p(jnp.abs(c) * MU) / LOG1P_MU)
            scaled = (enc + 1.0) * ((N_BINS - 1) / 2.0)
            binf = jnp.clip(jnp.round(scaled), 0.0, float(N_BINS - 1))
            z = binf * (2.0 / (N_BINS - 1)) - 1.0
            cq = (jnp.sign(z) * ((jnp.exp(jnp.abs(z) * LOG1P_MU) - 1.0) / MU)
                  * COEF_MAX)
            recon_q[s] = recon_q[s] + cq * (atoms_h[s][k] + a_l[s])
            toks[s].append(sels[s][k] * N_BINS + binf.astype(jnp.int32))
    sse = None
    zqs, tokcat = [], []
    for s in S:
        d = recon_q[s] - xs[s]
        sse_s = jnp.sum(d * d)
        sse = sse_s if sse is None else sse + sse_s
        zqs.append(xs[s] + (recon_q[s] - xs[s]))
        tokcat.append(jnp.concatenate(toks[s], axis=0))
    return zqs, tokcat, sse


def _omp_body(nb, denom, nh,
              x_ref, dict_ref,
              zq_ref, tok_ref, loss_ref,
              dnh_ref, dnl_ref, gh_ref, gm_ref):
    pid = pl.program_id(0)

    @pl.when(pid == 0)
    def _():
        d = dict_ref[...]
        n = jnp.sqrt(jnp.sum(d * d, axis=0, keepdims=True))
        dn = d / jnp.maximum(n, EPS)
        dn_h = dn.astype(BF16)
        dnh_ref[...] = dn_h
        dnl_ref[...] = (dn - dn_h.astype(F32)).astype(BF16)
        g = _mm(dn_h, dn_h, (((0,), (0,))))
        g_h = g.astype(BF16)
        gh_ref[...] = g_h
        gm_ref[...] = (g - g_h.astype(F32)).astype(BF16)

    x = x_ref[0]
    bt = x.shape[1]
    hw = bt // nh
    dn_h = dnh_ref[...]
    dn_l = dnl_ref[...]
    gh = gh_ref[...]
    gm = gm_ref[...]
    xs = [x[:, h * hw:(h + 1) * hw] for h in range(nh)]
    zqs, tokcat, sse = _omp_slabs(dn_h, dn_l, gh, gm, xs)
    for h in range(nh):
        zq_ref[0, :, h * hw:(h + 1) * hw] = zqs[h]
        tok_ref[:, h * hw:(h + 1) * hw] = tokcat[h]
    prev = jnp.where(pid == 0, jnp.zeros((1, 1), F32), loss_ref[...])
    tot = prev + sse
    loss_ref[...] = jnp.where(pid == nb - 1,
                              tot * ((1.0 + COMMIT) / denom), tot)


def kernel(z_e, dictionary):
    Bz, C, H, W = z_e.shape
    HW = H * W
    total = Bz * HW
    Bt = min(1024, HW)
    nh = 4
    pb = HW // Bt
    nb = total // Bt
    x3 = z_e.reshape(Bz, C, HW)
    body = functools.partial(_omp_body, nb, float(z_e.size), nh)
    zq3, tok, loss = pl.pallas_call(
        body,
        grid=(nb,),
        in_specs=[
            pl.BlockSpec((1, C, Bt), lambda i: (i // pb, 0, i % pb)),
            pl.BlockSpec((C, NUM_EMB), lambda i: (0, 0)),
        ],
        out_specs=[
            pl.BlockSpec((1, C, Bt), lambda i: (i // pb, 0, i % pb)),
            pl.BlockSpec((SPARSITY, Bt), lambda i: (0, i)),
            pl.BlockSpec((1, 1), lambda i: (0, 0)),
        ],
        out_shape=[
            jax.ShapeDtypeStruct((Bz, C, HW), jnp.float32),
            jax.ShapeDtypeStruct((SPARSITY, total), jnp.int32),
            jax.ShapeDtypeStruct((1, 1), jnp.float32),
        ],
        scratch_shapes=[
            pltpu.VMEM((C, NUM_EMB), BF16),
            pltpu.VMEM((C, NUM_EMB), BF16),
            pltpu.VMEM((NUM_EMB, NUM_EMB), BF16),
            pltpu.VMEM((NUM_EMB, NUM_EMB), BF16),
        ],
    )(x3, dictionary)
    z_q_ste = zq3.reshape(Bz, C, H, W)
    tokens = tok.T.reshape(Bz, H, W, SPARSITY)
    return z_q_ste, loss[0, 0], tokens


# speedup vs baseline: 93.0061x; 1.0692x over previous
import functools
import math

import jax
import jax.numpy as jnp
from jax.experimental import pallas as pl
from jax.experimental.pallas import tpu as pltpu

NUM_EMB = 1024
SPARSITY = 4
N_BINS = 16
COEF_MAX = 3.0
MU = 50.0
COMMIT = 0.25
EPS = 1e-10
LOG1P_MU = math.log1p(MU)
BF16 = jnp.bfloat16
F32 = jnp.float32


def _mm(a, b, dims):
    return jax.lax.dot_general(a, b, (dims, ((), ())),
                               preferred_element_type=F32)


def _ldl_solve(gram, rhs):
    m = len(rhs)
    L = {}
    Dd = []
    for j in range(m):
        dj = gram[(j, j)] + 1e-8
        for p in range(j):
            dj = dj - L[(j, p)] * L[(j, p)] * Dd[p]
        Dd.append(dj)
        inv_dj = 1.0 / dj
        for i in range(j + 1, m):
            s = gram[(i, j)]
            for p in range(j):
                s = s - L[(i, p)] * L[(j, p)] * Dd[p]
            L[(i, j)] = s * inv_dj
    y = []
    for i in range(m):
        s = rhs[i]
        for j in range(i):
            s = s - L[(i, j)] * y[j]
        y.append(s)
    z = [y[i] / Dd[i] for i in range(m)]
    x = [None] * m
    for i in reversed(range(m)):
        s = z[i]
        for j in range(i + 1, m):
            s = s - L[(j, i)] * x[j]
        x[i] = s
    return x


def _omp_slabs(dstk, gstk, xs):
    ns = len(xs)
    S = range(ns)
    bt = xs[0].shape[1]
    C = xs[0].shape[0]
    dn_h = dstk[:C]
    x_bf = [x.astype(BF16) for x in xs]
    alpha0 = [_mm(dn_h, xb, (((0,), (0,)))) for xb in x_bf]
    x_bf32 = [xb.astype(F32) for xb in x_bf]
    iota_n = jax.lax.broadcasted_iota(jnp.int32, (NUM_EMB, bt), 0)
    masked = [jnp.zeros((NUM_EMB, bt), dtype=jnp.bool_) for _ in S]
    corr = list(alpha0)
    onehots = [[] for _ in S]
    gcols = [[] for _ in S]
    atoms_h = [[] for _ in S]
    atoms_l = [[] for _ in S]
    rhs = [[] for _ in S]
    sels = [[] for _ in S]
    gram = [{} for _ in S]
    coeffs = [None] * ns
    for k in range(SPARSITY):
        absc = [jnp.where(masked[s], -1.0, jnp.abs(corr[s])) for s in S]
        mx = [jnp.max(absc[s], axis=0, keepdims=True) for s in S]
        sel = [jnp.min(jnp.where(absc[s] == mx[s], iota_n, NUM_EMB),
                       axis=0, keepdims=True) for s in S]
        onehot = [iota_n == sel[s] for s in S]
        oh = [onehot[s].astype(BF16) for s in S]
        a_b = [_mm(dstk, oh[s], (((1,), (0,)))) for s in S]
        a_h = [a_b[s][:C] for s in S]
        for s in S:
            masked[s] = jnp.logical_or(masked[s], onehot[s])
            sels[s].append(sel[s])
            onehots[s].append(onehot[s])
            atoms_h[s].append(a_h[s])
            atoms_l[s].append(a_b[s][C:])
            rhs[s].append(jnp.sum(a_h[s] * x_bf32[s], axis=0, keepdims=True))
            for j in range(k + 1):
                gram[s][(k, j)] = jnp.sum(a_h[s] * atoms_h[s][j],
                                          axis=0, keepdims=True)
        coeffs = [_ldl_solve(gram[s], rhs[s]) for s in S]
        if k < SPARSITY - 1:
            gcol_b = [_mm(gstk, oh[s], (((1,), (0,)))) for s in S]
            for s in S:
                gcols[s].append(gcol_b[s][:NUM_EMB] + gcol_b[s][NUM_EMB:])
                delta = coeffs[s][0] * gcols[s][0]
                for j in range(1, k + 1):
                    delta = delta + coeffs[s][j] * gcols[s][j]
                corr[s] = alpha0[s] - delta
    toks = [[] for _ in S]
    recon_q = [jnp.zeros_like(xs[s]) for s in S]
    for k in range(SPARSITY):
        for s in S:
            c = jnp.clip(coeffs[s][k], -COEF_MAX, COEF_MAX) / COEF_MAX
            enc = jnp.sign(c) * (jnp.log1p(jnp.abs(c) * MU) / LOG1P_MU)
            scaled = (enc + 1.0) * ((N_BINS - 1) / 2.0)
            binf = jnp.clip(jnp.round(scaled), 0.0, float(N_BINS - 1))
            z = binf * (2.0 / (N_BINS - 1)) - 1.0
            cq = (jnp.sign(z) * ((jnp.exp(jnp.abs(z) * LOG1P_MU) - 1.0) / MU)
                  * COEF_MAX)
            recon_q[s] = recon_q[s] + cq * (atoms_h[s][k] + atoms_l[s][k])
            toks[s].append(sels[s][k] * N_BINS + binf.astype(jnp.int32))
    sse = None
    zqs, tokcat = [], []
    for s in S:
        d = recon_q[s] - xs[s]
        sse_s = jnp.sum(d * d)
        sse = sse_s if sse is None else sse + sse_s
        zqs.append(xs[s] + (recon_q[s] - xs[s]))
        tokcat.append(jnp.concatenate(toks[s], axis=0))
    return zqs, tokcat, sse


def _omp_body(nb, denom, nh,
              x_ref, dict_ref,
              zq_ref, tok_ref, loss_ref,
              dstk_ref, gstk_ref):
    pid = pl.program_id(0)
    C = x_ref.shape[1]

    @pl.when(pid == 0)
    def _():
        d = dict_ref[...]
        n = jnp.sqrt(jnp.sum(d * d, axis=0, keepdims=True))
        dn = d / jnp.maximum(n, EPS)
        dn_h = dn.astype(BF16)
        dstk_ref[:C] = dn_h
        dstk_ref[C:] = (dn - dn_h.astype(F32)).astype(BF16)
        g = _mm(dn_h, dn_h, (((0,), (0,))))
        g_h = g.astype(BF16)
        gstk_ref[:NUM_EMB] = g_h
        gstk_ref[NUM_EMB:] = (g - g_h.astype(F32)).astype(BF16)

    x = x_ref[0]
    bt = x.shape[1]
    hw = bt // nh
    xs = [x[:, h * hw:(h + 1) * hw] for h in range(nh)]
    zqs, tokcat, sse = _omp_slabs(dstk_ref[...], gstk_ref[...], xs)
    for h in range(nh):
        zq_ref[0, :, h * hw:(h + 1) * hw] = zqs[h]
        tok_ref[:, h * hw:(h + 1) * hw] = tokcat[h]
    prev = jnp.where(pid == 0, jnp.zeros((1, 1), F32), loss_ref[...])
    tot = prev + sse
    loss_ref[...] = jnp.where(pid == nb - 1,
                              tot * ((1.0 + COMMIT) / denom), tot)


def kernel(z_e, dictionary):
    Bz, C, H, W = z_e.shape
    HW = H * W
    total = Bz * HW
    Bt = min(1024, HW)
    nh = 4
    pb = HW // Bt
    nb = total // Bt
    x3 = z_e.reshape(Bz, C, HW)
    body = functools.partial(_omp_body, nb, float(z_e.size), nh)
    zq3, tok, loss = pl.pallas_call(
        body,
        grid=(nb,),
        in_specs=[
            pl.BlockSpec((1, C, Bt), lambda i: (i // pb, 0, i % pb)),
            pl.BlockSpec((C, NUM_EMB), lambda i: (0, 0)),
        ],
        out_specs=[
            pl.BlockSpec((1, C, Bt), lambda i: (i // pb, 0, i % pb)),
            pl.BlockSpec((SPARSITY, Bt), lambda i: (0, i)),
            pl.BlockSpec((1, 1), lambda i: (0, 0)),
        ],
        out_shape=[
            jax.ShapeDtypeStruct((Bz, C, HW), jnp.float32),
            jax.ShapeDtypeStruct((SPARSITY, total), jnp.int32),
            jax.ShapeDtypeStruct((1, 1), jnp.float32),
        ],
        scratch_shapes=[
            pltpu.VMEM((2 * C, NUM_EMB), BF16),
            pltpu.VMEM((2 * NUM_EMB, NUM_EMB), BF16),
        ],
    )(x3, dictionary)
    z_q_ste = zq3.reshape(Bz, C, H, W)
    tokens = tok.T.reshape(Bz, H, W, SPARSITY)
    return z_q_ste, loss[0, 0], tokens
